# baseline (speedup 1.0000x reference)
import functools

import jax
import jax.numpy as jnp
from jax import lax
from jax.experimental import pallas as pl
from jax.experimental.pallas import tpu as pltpu
from jax.experimental.pallas import tpu_sc as plsc

_NUM_CORES = 2
_NUM_SUBCORES = 16
_NUM_WORKERS = _NUM_CORES * _NUM_SUBCORES

_CHUNK = 32
_NBUF = 3


def _embed_kernel(b_per_w, n_chunks, table_hbm, labels_hbm, out_hbm,
                  idx_v, rows_v, gsems, wsems):
  wid = lax.axis_index("s") * _NUM_CORES + lax.axis_index("c")
  base = wid * b_per_w
  pltpu.sync_copy(labels_hbm.at[pl.ds(base, b_per_w)], idx_v)

  def gather(c, slot):
    return pltpu.make_async_copy(
        table_hbm.at[idx_v.at[pl.ds(c * _CHUNK, _CHUNK)]],
        rows_v.at[slot], gsems.at[slot])

  def writeback(c, slot):
    return pltpu.make_async_copy(
        rows_v.at[slot], out_hbm.at[pl.ds(base + c * _CHUNK, _CHUNK)],
        wsems.at[slot])

  for c in range(_NBUF):
    gather(c, c).start()
  for c in range(n_chunks):
    slot = c % _NBUF
    gather(c, slot).wait()
    writeback(c, slot).start()
    writeback(c, slot).wait()
    if c + _NBUF < n_chunks:
      gather(c + _NBUF, slot).start()


def kernel(labels, table):
  batch = labels.shape[0]
  hidden = table.shape[1]
  b_per_w = batch // _NUM_WORKERS
  n_chunks = b_per_w // _CHUNK

  mesh = plsc.VectorSubcoreMesh(
      core_axis_name="c", subcore_axis_name="s",
      num_cores=_NUM_CORES, num_subcores=_NUM_SUBCORES)

  run = pl.kernel(
      functools.partial(_embed_kernel, b_per_w, n_chunks),
      out_type=jax.ShapeDtypeStruct((batch, hidden), jnp.float32),
      mesh=mesh,
      scratch_types=[
          pltpu.VMEM((b_per_w,), jnp.int32),
          pltpu.VMEM((_NBUF, _CHUNK, hidden), jnp.float32),
          pltpu.SemaphoreType.DMA((_NBUF,)),
          pltpu.SemaphoreType.DMA((_NBUF,)),
      ],
  )
  out = run(table, labels.astype(jnp.int32))
  return out[:, None, :]


# speedup vs baseline: 8.9982x; 8.9982x over previous
import functools

import jax
import jax.numpy as jnp
from jax import lax
from jax.experimental import pallas as pl
from jax.experimental.pallas import tpu as pltpu
from jax.experimental.pallas import tpu_sc as plsc

_NUM_CORES = 2
_NUM_SUBCORES = 16
_NUM_WORKERS = _NUM_CORES * _NUM_SUBCORES

_CHUNK = 32
_NSEM = 4


def _embed_kernel(b_per_w, n_chunks, table_hbm, labels_hbm, out_hbm,
                  lab_v, idx_s, bigbuf, sems):
  wid = lax.axis_index("s") * _NUM_CORES + lax.axis_index("c")
  base = wid * b_per_w
  pltpu.sync_copy(labels_hbm.at[pl.ds(base, b_per_w)], lab_v)

  sub = bigbuf.shape[1]
  pltpu.sync_copy(table_hbm.at[pl.ds(0, 1)], bigbuf.at[pl.ds(0, 1)])
  pltpu.sync_copy(table_hbm.at[pl.ds(1, 1)], bigbuf.at[pl.ds(_CHUNK, 1)])

  def _replicate(r, _):
    for s in range(sub):
      for j in range(128 // 16):
        bigbuf[r, s, pl.ds(16 * j, 16)] = bigbuf[0, s, pl.ds(16 * j, 16)]
        bigbuf[_CHUNK + r, s, pl.ds(16 * j, 16)] = (
            bigbuf[_CHUNK, s, pl.ds(16 * j, 16)])
    return ()

  lax.fori_loop(1, _CHUNK, _replicate, ())

  lane = lax.iota(jnp.int32, 16)
  scatters = []
  for c in range(n_chunks):
    laba = lab_v[pl.ds(c * _CHUNK, 16)]
    labb = lab_v[pl.ds(c * _CHUNK + 16, 16)]
    m0a = laba == 0
    m0b = labb == 0
    i0a = m0a.astype(jnp.int32)
    i0b = m0b.astype(jnp.int32)
    n0a = jnp.sum(i0a)
    n0b = jnp.sum(i0b)
    n0 = n0a + n0b
    e0a = plsc.cumsum(i0a) - i0a
    e0b = plsc.cumsum(i0b) - i0b
    e1a = plsc.cumsum(1 - i0a) - (1 - i0a)
    e1b = plsc.cumsum(1 - i0b) - (1 - i0b)
    rank_a = jnp.where(m0a, e0a, n0 + e1a)
    rank_b = jnp.where(m0b, n0a + e0b, n0 + (16 - n0a) + e1b)
    pos_a = base + c * _CHUNK + lane
    pos_b = pos_a + 16
    plsc.store_scatter(idx_s.at[c], [rank_a], pos_a)
    plsc.store_scatter(idx_s.at[c], [rank_b], pos_b)
    cp = pltpu.make_async_copy(
        bigbuf.at[pl.ds(_CHUNK - n0, _CHUNK)],
        out_hbm.at[idx_s.at[c]],
        sems.at[c % _NSEM])
    cp.start()
    scatters.append(cp)
  for cp in scatters:
    cp.wait()


def kernel(labels, table):
  batch = labels.shape[0]
  hidden = table.shape[1]
  b_per_w = batch // _NUM_WORKERS
  n_chunks = b_per_w // _CHUNK

  mesh = plsc.VectorSubcoreMesh(
      core_axis_name="c", subcore_axis_name="s",
      num_cores=_NUM_CORES, num_subcores=_NUM_SUBCORES)

  sub = hidden // 128
  run = pl.kernel(
      functools.partial(_embed_kernel, b_per_w, n_chunks),
      out_type=jax.ShapeDtypeStruct((batch, sub, 128), jnp.float32),
      mesh=mesh,
      compiler_params=pltpu.CompilerParams(use_tc_tiling_on_sc=False, needs_layout_passes=False),
      scratch_types=[
          pltpu.VMEM((b_per_w,), jnp.int32),
          pltpu.VMEM((n_chunks, _CHUNK), jnp.int32),
          pltpu.VMEM((2 * _CHUNK, sub, 128), jnp.float32),
          pltpu.SemaphoreType.DMA((_NSEM,)),
      ],
  )
  out = run(table.reshape(2, sub, 128), labels.astype(jnp.int32))
  return out.reshape(batch, 1, hidden)


# speedup vs baseline: 9.0162x; 1.0020x over previous
import functools

import jax
import jax.numpy as jnp
from jax import lax
from jax.experimental import pallas as pl
from jax.experimental.pallas import tpu as pltpu
from jax.experimental.pallas import tpu_sc as plsc

_NUM_CORES = 2
_NUM_SUBCORES = 16
_NUM_WORKERS = _NUM_CORES * _NUM_SUBCORES

_CHUNK = 32
_NSEM = 4


def _embed_kernel(b_per_w, n_chunks, table_hbm, labels_hbm, out_hbm,
                  lab_v, idx_s, bigbuf, sems):
  wid = lax.axis_index("s") * _NUM_CORES + lax.axis_index("c")
  base = wid * b_per_w
  pltpu.sync_copy(labels_hbm.at[pl.ds(base, b_per_w)], lab_v)

  sub = bigbuf.shape[1]
  pltpu.sync_copy(table_hbm.at[pl.ds(0, 1)], bigbuf.at[pl.ds(0, 1)])
  pltpu.sync_copy(table_hbm.at[pl.ds(1, 1)], bigbuf.at[pl.ds(_CHUNK, 1)])

  def _replicate(r, _):
    for s in range(sub):
      for j in range(128 // 16):
        bigbuf[r, s, pl.ds(16 * j, 16)] = bigbuf[0, s, pl.ds(16 * j, 16)]
        bigbuf[_CHUNK + r, s, pl.ds(16 * j, 16)] = (
            bigbuf[_CHUNK, s, pl.ds(16 * j, 16)])
    return ()

  lax.fori_loop(1, _CHUNK, _replicate, ())

  lane = lax.iota(jnp.int32, 16)
  scatters = []
  for c in range(n_chunks):
    laba = lab_v[pl.ds(c * _CHUNK, 16)]
    labb = lab_v[pl.ds(c * _CHUNK + 16, 16)]
    m0a = laba == 0
    m0b = labb == 0
    i0a = m0a.astype(jnp.int32)
    i0b = m0b.astype(jnp.int32)
    n0a = jnp.sum(i0a)
    n0b = jnp.sum(i0b)
    n0 = n0a + n0b
    e0a = plsc.cumsum(i0a) - i0a
    e0b = plsc.cumsum(i0b) - i0b
    e1a = plsc.cumsum(1 - i0a) - (1 - i0a)
    e1b = plsc.cumsum(1 - i0b) - (1 - i0b)
    rank_a = jnp.where(m0a, e0a, n0 + e1a)
    rank_b = jnp.where(m0b, n0a + e0b, n0 + (16 - n0a) + e1b)
    pos_a = base + c * _CHUNK + lane
    pos_b = pos_a + 16
    plsc.store_scatter(idx_s.at[c], [rank_a], pos_a)
    plsc.store_scatter(idx_s.at[c], [rank_b], pos_b)
    cp = pltpu.make_async_copy(
        bigbuf.at[pl.ds(_CHUNK - n0, _CHUNK)],
        out_hbm.at[idx_s.at[c]],
        sems.at[c % _NSEM])
    cp.start()
    scatters.append(cp)
  for cp in scatters:
    cp.wait()


def kernel(labels, table):
  batch = labels.shape[0]
  hidden = table.shape[1]
  b_per_w = batch // _NUM_WORKERS
  n_chunks = b_per_w // _CHUNK

  mesh = plsc.VectorSubcoreMesh(
      core_axis_name="c", subcore_axis_name="s",
      num_cores=_NUM_CORES, num_subcores=_NUM_SUBCORES)

  sub = hidden // 128
  run = pl.kernel(
      functools.partial(_embed_kernel, b_per_w, n_chunks),
      out_type=jax.ShapeDtypeStruct((batch, sub, 128), jnp.float32),
      mesh=mesh,
      compiler_params=pltpu.CompilerParams(use_tc_tiling_on_sc=False, needs_layout_passes=False, skip_device_barrier=True),
      scratch_types=[
          pltpu.VMEM((b_per_w,), jnp.int32),
          pltpu.VMEM((n_chunks, _CHUNK), jnp.int32),
          pltpu.VMEM((2 * _CHUNK, sub, 128), jnp.float32),
          pltpu.SemaphoreType.DMA((_NSEM,)),
      ],
  )
  out = run(table.reshape(2, sub, 128), labels.astype(jnp.int32))
  return out.reshape(batch, 1, hidden)


# speedup vs baseline: 9.3277x; 1.0345x over previous
import functools

import jax
import jax.numpy as jnp
from jax import lax
from jax.experimental import pallas as pl
from jax.experimental.pallas import tpu as pltpu
from jax.experimental.pallas import tpu_sc as plsc

_NUM_CORES = 2
_NUM_SUBCORES = 16
_NUM_WORKERS = _NUM_CORES * _NUM_SUBCORES

_CHUNK = 32
_NSEM = 4


def _embed_kernel(b_per_w, n_chunks, table_hbm, labels_hbm, out_hbm,
                  lab_v, idx_s, bigbuf, sems):
  wid = lax.axis_index("s") * _NUM_CORES + lax.axis_index("c")
  base = wid * b_per_w
  pltpu.sync_copy(labels_hbm.at[pl.ds(base, b_per_w)], lab_v)

  sub = bigbuf.shape[1]
  pltpu.sync_copy(table_hbm.at[pl.ds(0, 1)], bigbuf.at[pl.ds(0, 1)])
  pltpu.sync_copy(table_hbm.at[pl.ds(1, 1)], bigbuf.at[pl.ds(_CHUNK, 1)])

  for s in range(sub):
    for j in range(128 // 16):
      v0 = bigbuf[0, s, pl.ds(16 * j, 16)]
      v1 = bigbuf[_CHUNK, s, pl.ds(16 * j, 16)]
      for r in range(1, _CHUNK):
        bigbuf[r, s, pl.ds(16 * j, 16)] = v0
        bigbuf[_CHUNK + r, s, pl.ds(16 * j, 16)] = v1

  lane = lax.iota(jnp.int32, 16)
  scatters = []
  for c in range(n_chunks):
    laba = lab_v[pl.ds(c * _CHUNK, 16)]
    labb = lab_v[pl.ds(c * _CHUNK + 16, 16)]
    m0a = laba == 0
    m0b = labb == 0
    i0a = m0a.astype(jnp.int32)
    i0b = m0b.astype(jnp.int32)
    n0a = jnp.sum(i0a)
    n0b = jnp.sum(i0b)
    n0 = n0a + n0b
    e0a = plsc.cumsum(i0a) - i0a
    e0b = plsc.cumsum(i0b) - i0b
    e1a = plsc.cumsum(1 - i0a) - (1 - i0a)
    e1b = plsc.cumsum(1 - i0b) - (1 - i0b)
    rank_a = jnp.where(m0a, e0a, n0 + e1a)
    rank_b = jnp.where(m0b, n0a + e0b, n0 + (16 - n0a) + e1b)
    pos_a = base + c * _CHUNK + lane
    pos_b = pos_a + 16
    plsc.store_scatter(idx_s.at[c], [rank_a], pos_a)
    plsc.store_scatter(idx_s.at[c], [rank_b], pos_b)
    cp = pltpu.make_async_copy(
        bigbuf.at[pl.ds(_CHUNK - n0, _CHUNK)],
        out_hbm.at[idx_s.at[c]],
        sems.at[c % _NSEM])
    cp.start()
    scatters.append(cp)
  for cp in scatters:
    cp.wait()


def kernel(labels, table):
  batch = labels.shape[0]
  hidden = table.shape[1]
  b_per_w = batch // _NUM_WORKERS
  n_chunks = b_per_w // _CHUNK

  mesh = plsc.VectorSubcoreMesh(
      core_axis_name="c", subcore_axis_name="s",
      num_cores=_NUM_CORES, num_subcores=_NUM_SUBCORES)

  sub = hidden // 128
  run = pl.kernel(
      functools.partial(_embed_kernel, b_per_w, n_chunks),
      out_type=jax.ShapeDtypeStruct((batch, sub, 128), jnp.float32),
      mesh=mesh,
      compiler_params=pltpu.CompilerParams(use_tc_tiling_on_sc=False, needs_layout_passes=False, skip_device_barrier=True),
      scratch_types=[
          pltpu.VMEM((b_per_w,), jnp.int32),
          pltpu.VMEM((n_chunks, _CHUNK), jnp.int32),
          pltpu.VMEM((2 * _CHUNK, sub, 128), jnp.float32),
          pltpu.SemaphoreType.DMA((_NSEM,)),
      ],
  )
  out = run(table.reshape(2, sub, 128), labels.astype(jnp.int32))
  return out.reshape(batch, 1, hidden)


# speedup vs baseline: 11.0421x; 1.1838x over previous
import functools

import jax
import jax.numpy as jnp
from jax import lax
from jax.experimental import pallas as pl
from jax.experimental.pallas import tpu as pltpu
from jax.experimental.pallas import tpu_sc as plsc

_NUM_CORES = 2
_NUM_SUBCORES = 16
_NUM_WORKERS = _NUM_CORES * _NUM_SUBCORES

_CHUNK = 16
_NSEM = 4


def _embed_kernel(b_per_w, n_chunks, table_hbm, labels_hbm, out_hbm,
                  lab_v, idx_s, bigbuf, sems, lsem):
  wid = lax.axis_index("s") * _NUM_CORES + lax.axis_index("c")
  base = wid * b_per_w
  sub = bigbuf.shape[1]

  lab_cp = pltpu.make_async_copy(
      labels_hbm.at[pl.ds(base, b_per_w)], lab_v, lsem)
  lab_cp.start()
  r0 = pltpu.make_async_copy(
      table_hbm.at[pl.ds(0, 1)], bigbuf.at[pl.ds(0, 1)], lsem)
  r1 = pltpu.make_async_copy(
      table_hbm.at[pl.ds(1, 1)], bigbuf.at[pl.ds(_CHUNK, 1)], lsem)
  r0.start()
  r1.start()
  r0.wait()
  r1.wait()

  for s in range(sub):
    for j in range(128 // 16):
      v0 = bigbuf[0, s, pl.ds(16 * j, 16)]
      v1 = bigbuf[_CHUNK, s, pl.ds(16 * j, 16)]
      for r in range(1, _CHUNK):
        bigbuf[r, s, pl.ds(16 * j, 16)] = v0
        bigbuf[_CHUNK + r, s, pl.ds(16 * j, 16)] = v1
  lab_cp.wait()

  lane = lax.iota(jnp.int32, 16)

  def chunk_body(c, _):
    lab = lab_v[pl.ds(c * _CHUNK, 16)]
    m0 = lab == 0
    i0 = m0.astype(jnp.int32)
    n0 = jnp.sum(i0)
    e0 = plsc.cumsum(i0) - i0
    e1 = plsc.cumsum(1 - i0) - (1 - i0)
    rank = jnp.where(m0, e0, n0 + e1)
    pos = base + c * _CHUNK + lane
    plsc.store_scatter(idx_s.at[c], [rank], pos)
    pltpu.make_async_copy(
        bigbuf.at[pl.ds(_CHUNK - n0, _CHUNK)],
        out_hbm.at[idx_s.at[c]],
        sems.at[c % _NSEM]).start()
    return ()

  lax.fori_loop(0, n_chunks, chunk_body, ())

  def drain_body(c, _):
    pltpu.make_async_copy(
        bigbuf.at[pl.ds(0, _CHUNK)],
        out_hbm.at[idx_s.at[c]],
        sems.at[c % _NSEM]).wait()
    return ()

  lax.fori_loop(0, n_chunks, drain_body, ())


def kernel(labels, table):
  batch = labels.shape[0]
  hidden = table.shape[1]
  b_per_w = batch // _NUM_WORKERS
  n_chunks = b_per_w // _CHUNK

  mesh = plsc.VectorSubcoreMesh(
      core_axis_name="c", subcore_axis_name="s",
      num_cores=_NUM_CORES, num_subcores=_NUM_SUBCORES)

  sub = hidden // 128
  run = pl.kernel(
      functools.partial(_embed_kernel, b_per_w, n_chunks),
      out_type=jax.ShapeDtypeStruct((batch, sub, 128), jnp.float32),
      mesh=mesh,
      compiler_params=pltpu.CompilerParams(
          use_tc_tiling_on_sc=False, needs_layout_passes=False,
          skip_device_barrier=True),
      scratch_types=[
          pltpu.VMEM((b_per_w,), jnp.int32),
          pltpu.VMEM((n_chunks, _CHUNK), jnp.int32),
          pltpu.VMEM((2 * _CHUNK, sub, 128), jnp.float32),
          pltpu.SemaphoreType.DMA((_NSEM,)),
          pltpu.SemaphoreType.DMA,
      ],
  )
  out = run(table.reshape(2, sub, 128), labels.astype(jnp.int32))
  return out.reshape(batch, 1, hidden)
